# Initial kernel scaffold; baseline (speedup 1.0000x reference)
import jax
import jax.numpy as jnp
from jax.experimental import pallas as pl


def kernel(z, codebook):
    raise NotImplementedError("write your pallas kernel here")


# speedup vs baseline: 1.0818x; 1.0818x over previous
import jax
import jax.numpy as jnp
from jax.experimental import pallas as pl

NUM_CODES = 1024
LATENT_DIM = 256
BETA = 0.25


def _vq_body(zp_ref, cb_ref, cbt_ref, zq_ref, idx_ref, part_ref):
    zp = zp_ref[0]
    cb = cb_ref[...]
    cbt = cbt_ref[...]

    mm = jnp.dot(zp, cbt, preferred_element_type=jnp.float32)
    zn = jnp.sum(zp * zp, axis=1)
    cn = jnp.sum(cb * cb, axis=1)
    dist = (zn[:, None] + cn[None, :]) - 2.0 * mm

    m = jnp.min(dist, axis=1)
    j_row = jax.lax.broadcasted_iota(jnp.int32, dist.shape, 1)
    idx = jnp.min(jnp.where(dist == m[:, None], j_row, NUM_CODES), axis=1)

    j_col = jax.lax.broadcasted_iota(jnp.int32, (NUM_CODES, zp.shape[0]), 0)
    onehot = (j_col == idx[None, :]).astype(jnp.float32)
    zq = jnp.dot(cbt, onehot, preferred_element_type=jnp.float32,
                 precision=jax.lax.Precision.HIGHEST)

    zq_ref[0] = zq
    idx_ref[0, 0] = idx
    part_ref[0, 0] = m


def kernel(z, codebook):
    B, C, H, W = z.shape
    P = H * W
    zp = z.transpose(0, 2, 3, 1).reshape(B, P, C)
    cbt = codebook.T

    zq3, idx3, parts = pl.pallas_call(
        _vq_body,
        grid=(B,),
        in_specs=[
            pl.BlockSpec((1, P, C), lambda b: (b, 0, 0)),
            pl.BlockSpec((NUM_CODES, C), lambda b: (0, 0)),
            pl.BlockSpec((C, NUM_CODES), lambda b: (0, 0)),
        ],
        out_specs=[
            pl.BlockSpec((1, C, P), lambda b: (b, 0, 0)),
            pl.BlockSpec((1, 1, P), lambda b: (b, 0, 0)),
            pl.BlockSpec((1, 1, P), lambda b: (b, 0, 0)),
        ],
        out_shape=[
            jax.ShapeDtypeStruct((B, C, P), jnp.float32),
            jax.ShapeDtypeStruct((B, 1, P), jnp.int32),
            jax.ShapeDtypeStruct((B, 1, P), jnp.float32),
        ],
    )(zp, codebook, cbt)

    zq_out = zq3.reshape(B, C, H, W)
    dist_indices = idx3.reshape(B * P)
    loss = (1.0 - BETA) * (jnp.sum(parts) / (B * P * C))
    return zq_out, dist_indices, loss


# speedup vs baseline: 1.6346x; 1.5110x over previous
import jax
import jax.numpy as jnp
from jax.experimental import pallas as pl

NUM_CODES = 1024
LATENT_DIM = 256
BETA = 0.25


def _vq_body(zp_ref, cb_ref, cbt_ref, zq_ref, idx_ref, part_ref):
    zp = zp_ref[0]
    cb = cb_ref[...]
    cbt = cbt_ref[...]

    mm = jnp.dot(zp, cbt, preferred_element_type=jnp.float32)
    zn = jnp.sum(zp * zp, axis=1)
    cn = jnp.sum(cb * cb, axis=1)
    dist = (zn[:, None] + cn[None, :]) - 2.0 * mm

    m = jnp.min(dist, axis=1)
    j_row = jax.lax.broadcasted_iota(jnp.int32, dist.shape, 1)
    idx = jnp.min(jnp.where(dist == m[:, None], j_row, NUM_CODES), axis=1)

    j_col = jax.lax.broadcasted_iota(jnp.int32, (NUM_CODES, zp.shape[0]), 0)
    onehot = (j_col == idx[None, :]).astype(jnp.float32)
    zq = jnp.dot(cbt, onehot, preferred_element_type=jnp.float32)

    zq_ref[0] = zq
    idx_ref[0, 0] = idx
    part_ref[0, 0] = m


def kernel(z, codebook):
    B, C, H, W = z.shape
    P = H * W
    zp = z.transpose(0, 2, 3, 1).reshape(B, P, C)
    cbt = codebook.T

    zq3, idx3, parts = pl.pallas_call(
        _vq_body,
        grid=(B,),
        in_specs=[
            pl.BlockSpec((1, P, C), lambda b: (b, 0, 0)),
            pl.BlockSpec((NUM_CODES, C), lambda b: (0, 0)),
            pl.BlockSpec((C, NUM_CODES), lambda b: (0, 0)),
        ],
        out_specs=[
            pl.BlockSpec((1, C, P), lambda b: (b, 0, 0)),
            pl.BlockSpec((1, 1, P), lambda b: (b, 0, 0)),
            pl.BlockSpec((1, 1, P), lambda b: (b, 0, 0)),
        ],
        out_shape=[
            jax.ShapeDtypeStruct((B, C, P), jnp.float32),
            jax.ShapeDtypeStruct((B, 1, P), jnp.int32),
            jax.ShapeDtypeStruct((B, 1, P), jnp.float32),
        ],
    )(zp, codebook, cbt)

    zq_out = zq3.reshape(B, C, H, W)
    dist_indices = idx3.reshape(B * P)
    loss = (1.0 - BETA) * (jnp.sum(parts) / (B * P * C))
    return zq_out, dist_indices, loss
